# initial kernel scaffold (unmeasured)
import jax
import jax.numpy as jnp
from jax import lax
from jax.experimental import pallas as pl
from jax.experimental.pallas import tpu as pltpu

N_DEV = 32
LOG2_N = 5
N_GLOBAL = 16384
EPS = 1e-5


def kernel(x, gamma, beta):
    m, n_per = x.shape

    def body(x_ref, g_ref, b_ref, o_ref, acc_ref, recv_ref, send_sems, recv_sems):
        my = lax.axis_index("i")

        xv = x_ref[:, :]
        acc_ref[0, :] = jnp.sum(xv, axis=1)
        acc_ref[1, :] = jnp.sum(xv * xv, axis=1)

        for k in range(LOG2_N):
            partner = my ^ (1 << k)
            rdma = pltpu.make_async_remote_copy(
                src_ref=acc_ref,
                dst_ref=recv_ref.at[k],
                send_sem=send_sems.at[k],
                recv_sem=recv_sems.at[k],
                device_id=(partner,),
                device_id_type=pl.DeviceIdType.MESH,
            )
            rdma.start()
            rdma.wait()
            acc_ref[:, :] = acc_ref[:, :] + recv_ref[k, :, :]

        mean = (acc_ref[0, :] / N_GLOBAL).reshape(m, 1)
        ex2 = (acc_ref[1, :] / N_GLOBAL).reshape(m, 1)
        inv = lax.rsqrt(ex2 - mean * mean + EPS)
        o_ref[:, :] = g_ref[0, :] * ((xv - mean) * inv) + b_ref[0, :]

    return pl.pallas_call(
        body,
        out_shape=jax.ShapeDtypeStruct((m, n_per), jnp.float32),
        in_specs=[
            pl.BlockSpec(memory_space=pltpu.VMEM),
            pl.BlockSpec(memory_space=pltpu.VMEM),
            pl.BlockSpec(memory_space=pltpu.VMEM),
        ],
        out_specs=pl.BlockSpec(memory_space=pltpu.VMEM),
        scratch_shapes=[
            pltpu.VMEM((2, m), jnp.float32),
            pltpu.VMEM((LOG2_N, 2, m), jnp.float32),
            pltpu.SemaphoreType.DMA((LOG2_N,)),
            pltpu.SemaphoreType.DMA((LOG2_N,)),
        ],
        compiler_params=pltpu.CompilerParams(collective_id=0),
    )(x, gamma.reshape(1, n_per), beta.reshape(1, n_per))


# baseline (device time: 29247 ns/iter reference)
import jax
import jax.numpy as jnp
from jax import lax
from jax.experimental import pallas as pl
from jax.experimental.pallas import tpu as pltpu

N_DEV = 32
LOG2_N = 5
N_GLOBAL = 16384
EPS = 1e-5


def kernel(x, gamma, beta):
    m, n_per = x.shape

    def body(x_ref, g_ref, b_ref, o_ref, acc_ref, recv_ref, send_sems, recv_sems):
        my = lax.axis_index("i")

        xv = x_ref[:, :]
        acc_ref[0, :] = jnp.sum(xv, axis=1)
        acc_ref[1, :] = jnp.sum(xv * xv, axis=1)

        for k in range(LOG2_N):
            partner = my ^ (1 << k)
            rdma = pltpu.make_async_remote_copy(
                src_ref=acc_ref,
                dst_ref=recv_ref.at[k],
                send_sem=send_sems.at[k],
                recv_sem=recv_sems.at[k],
                device_id=(partner,),
                device_id_type=pl.DeviceIdType.MESH,
            )
            rdma.start()
            rdma.wait()
            acc_ref[:, :] = acc_ref[:, :] + recv_ref[k, :, :]

        mean = (acc_ref[0, :] / N_GLOBAL).reshape(m, 1)
        ex2 = (acc_ref[1, :] / N_GLOBAL).reshape(m, 1)
        inv = lax.rsqrt(ex2 - mean * mean + EPS)
        o_ref[:, :] = g_ref[0, :] * ((xv - mean) * inv) + b_ref[0, :]

    return pl.pallas_call(
        body,
        out_shape=jax.ShapeDtypeStruct((m, n_per), jnp.float32),
        in_specs=[
            pl.BlockSpec(memory_space=pltpu.VMEM),
            pl.BlockSpec(memory_space=pltpu.VMEM),
            pl.BlockSpec(memory_space=pltpu.VMEM),
        ],
        out_specs=pl.BlockSpec(memory_space=pltpu.VMEM),
        scratch_shapes=[
            pltpu.VMEM((2, m), jnp.float32),
            pltpu.VMEM((LOG2_N, 2, m), jnp.float32),
            pltpu.SemaphoreType.DMA((LOG2_N,)),
            pltpu.SemaphoreType.DMA((LOG2_N,)),
        ],
    )(x, gamma.reshape(1, n_per), beta.reshape(1, n_per))


# device time: 26558 ns/iter; 1.1013x vs baseline; 1.1013x over previous
import jax
import jax.numpy as jnp
from jax import lax
from jax.experimental import pallas as pl
from jax.experimental.pallas import tpu as pltpu

N_DEV = 32
N_GLOBAL = 16384
EPS = 1e-5


def kernel(x, gamma, beta):
    m, n_per = x.shape

    def body(x_ref, g_ref, b_ref, o_ref, stats_ref, send_sems, recv_sems):
        my = lax.axis_index("i")

        xv = x_ref[:, :]
        stats_ref[0, 0, :] = jnp.sum(xv, axis=1)
        stats_ref[0, 1, :] = jnp.sum(xv * xv, axis=1)

        rdmas = []
        for d in range(1, N_DEV):
            peer = my ^ d
            rdma = pltpu.make_async_remote_copy(
                src_ref=stats_ref.at[0],
                dst_ref=stats_ref.at[d],
                send_sem=send_sems.at[d],
                recv_sem=recv_sems.at[d],
                device_id=(peer,),
                device_id_type=pl.DeviceIdType.MESH,
            )
            rdma.start()
            rdmas.append(rdma)

        for rdma in rdmas:
            rdma.wait_recv()

        total = jnp.sum(stats_ref[:, :, :], axis=0)
        mean = (total[0, :] / N_GLOBAL).reshape(m, 1)
        ex2 = (total[1, :] / N_GLOBAL).reshape(m, 1)
        inv = lax.rsqrt(ex2 - mean * mean + EPS)
        o_ref[:, :] = g_ref[0, :] * ((xv - mean) * inv) + b_ref[0, :]

        for rdma in rdmas:
            rdma.wait_send()

    return pl.pallas_call(
        body,
        out_shape=jax.ShapeDtypeStruct((m, n_per), jnp.float32),
        in_specs=[
            pl.BlockSpec(memory_space=pltpu.VMEM),
            pl.BlockSpec(memory_space=pltpu.VMEM),
            pl.BlockSpec(memory_space=pltpu.VMEM),
        ],
        out_specs=pl.BlockSpec(memory_space=pltpu.VMEM),
        scratch_shapes=[
            pltpu.VMEM((N_DEV, 2, m), jnp.float32),
            pltpu.SemaphoreType.DMA((N_DEV,)),
            pltpu.SemaphoreType.DMA((N_DEV,)),
        ],
    )(x, gamma.reshape(1, n_per), beta.reshape(1, n_per))


# device time: 18313 ns/iter; 1.5971x vs baseline; 1.4502x over previous
import jax
import jax.numpy as jnp
from jax import lax
from jax.experimental import pallas as pl
from jax.experimental.pallas import tpu as pltpu

N_DEV = 32
N_GLOBAL = 16384
EPS = 1e-5


def kernel(x, gamma, beta):
    m, n_per = x.shape

    def body(x_ref, g_ref, b_ref, o_ref, stats_ref, send_sems, recv_sems):
        my = lax.axis_index("i")

        barrier_sem = pltpu.get_barrier_semaphore()
        for d in range(1, N_DEV):
            pl.semaphore_signal(
                barrier_sem, inc=1,
                device_id=(my ^ d,), device_id_type=pl.DeviceIdType.MESH,
            )

        xv = x_ref[:, :]
        stats_ref[0, 0, :] = jnp.sum(xv, axis=1)
        stats_ref[0, 1, :] = jnp.sum(xv * xv, axis=1)

        pl.semaphore_wait(barrier_sem, N_DEV - 1)

        rdmas = []
        for d in range(1, N_DEV):
            peer = my ^ d
            rdma = pltpu.make_async_remote_copy(
                src_ref=stats_ref.at[0],
                dst_ref=stats_ref.at[d],
                send_sem=send_sems.at[d],
                recv_sem=recv_sems.at[d],
                device_id=(peer,),
                device_id_type=pl.DeviceIdType.MESH,
            )
            rdma.start()
            rdmas.append(rdma)

        for rdma in rdmas:
            rdma.wait_recv()

        total = jnp.sum(stats_ref[:, :, :], axis=0)
        mean = (total[0, :] / N_GLOBAL).reshape(m, 1)
        ex2 = (total[1, :] / N_GLOBAL).reshape(m, 1)
        inv = lax.rsqrt(ex2 - mean * mean + EPS)
        o_ref[:, :] = g_ref[0, :] * ((xv - mean) * inv) + b_ref[0, :]

        for rdma in rdmas:
            rdma.wait_send()

    return pl.pallas_call(
        body,
        out_shape=jax.ShapeDtypeStruct((m, n_per), jnp.float32),
        in_specs=[
            pl.BlockSpec(memory_space=pltpu.VMEM),
            pl.BlockSpec(memory_space=pltpu.VMEM),
            pl.BlockSpec(memory_space=pltpu.VMEM),
        ],
        out_specs=pl.BlockSpec(memory_space=pltpu.VMEM),
        scratch_shapes=[
            pltpu.VMEM((N_DEV, 2, m), jnp.float32),
            pltpu.SemaphoreType.DMA((N_DEV,)),
            pltpu.SemaphoreType.DMA((N_DEV,)),
        ],
        compiler_params=pltpu.CompilerParams(collective_id=0),
    )(x, gamma.reshape(1, n_per), beta.reshape(1, n_per))
